# baseline (speedup 1.0000x reference)
import functools

import jax
import jax.numpy as jnp
from jax import lax
from jax.experimental import pallas as pl
from jax.experimental.pallas import tpu as pltpu
from jax.experimental.pallas import tpu_sc as plsc

B = 16384
K = 64
NUM_WORKERS = 32
TRIPLES_PER_WORKER = B // NUM_WORKERS
CHUNKS = TRIPLES_PER_WORKER // 128
GROUPS = TRIPLES_PER_WORKER // 16


def _sqrt16(x):
    i = plsc.bitcast(x, jnp.int32)
    magic = jnp.full((16,), 0x5F3759DF, dtype=jnp.int32)
    y = plsc.bitcast(magic - (i >> 1), jnp.float32)
    half = jnp.full((16,), 0.5, dtype=jnp.float32)
    threehalf = jnp.full((16,), 1.5, dtype=jnp.float32)
    hx = half * x
    for _ in range(3):
        y = y * (threehalf - hx * y * y)
    return x * y


def _body(hs, ls, ts, emb_E, emb_R, out,
          idx_h, idx_l, idx_t, h_rows, l_rows, t_rows, out_v, sem):
    wid = lax.axis_index("s") * 2 + lax.axis_index("c")
    base_chunk = wid * CHUNKS

    pltpu.sync_copy(hs.at[pl.ds(base_chunk, CHUNKS)], idx_h)
    pltpu.sync_copy(ls.at[pl.ds(base_chunk, CHUNKS)], idx_l)
    pltpu.sync_copy(ts.at[pl.ds(base_chunk, CHUNKS)], idx_t)

    copies = []
    for c in range(CHUNKS):
        dst = pl.ds(c * 128, 128)
        copies.append(pltpu.async_copy(emb_E.at[idx_h.at[c]], h_rows.at[dst], sem))
        copies.append(pltpu.async_copy(emb_R.at[idx_l.at[c]], l_rows.at[dst], sem))
        copies.append(pltpu.async_copy(emb_E.at[idx_t.at[c]], t_rows.at[dst], sem))
    for cp in copies:
        cp.wait()

    lane = lax.iota(jnp.int32, 16)

    def group_body(g, carry):
        row = g * 16 + lane
        acc = jnp.zeros((16,), jnp.float32)
        for j in range(K):
            col = jnp.full((16,), j, dtype=jnp.int32)
            hv = plsc.load_gather(h_rows, [row, col])
            lv = plsc.load_gather(l_rows, [row, col])
            tv = plsc.load_gather(t_rows, [row, col])
            d = hv + lv - tv
            acc = acc + d * d
        plsc.store_scatter(out_v, [row], _sqrt16(acc))
        return carry

    lax.fori_loop(0, GROUPS, group_body, 0)

    pltpu.sync_copy(out_v, out.at[pl.ds(wid * TRIPLES_PER_WORKER,
                                        TRIPLES_PER_WORKER)])


@functools.partial(jax.jit, donate_argnums=())
def _transe(hs, ls, ts, emb_E, emb_R):
    mesh = plsc.VectorSubcoreMesh(core_axis_name="c", subcore_axis_name="s")
    f = functools.partial(
        pl.kernel,
        out_type=jax.ShapeDtypeStruct((B,), jnp.float32),
        mesh=mesh,
        compiler_params=pltpu.CompilerParams(
            needs_layout_passes=False, use_tc_tiling_on_sc=False),
        scratch_types=[
            pltpu.VMEM((CHUNKS, 128), jnp.int32),
            pltpu.VMEM((CHUNKS, 128), jnp.int32),
            pltpu.VMEM((CHUNKS, 128), jnp.int32),
            pltpu.VMEM((TRIPLES_PER_WORKER, K), jnp.float32),
            pltpu.VMEM((TRIPLES_PER_WORKER, K), jnp.float32),
            pltpu.VMEM((TRIPLES_PER_WORKER, K), jnp.float32),
            pltpu.VMEM((TRIPLES_PER_WORKER,), jnp.float32),
            pltpu.SemaphoreType.DMA,
        ],
    )(_body)
    return f(hs, ls, ts, emb_E, emb_R)


def kernel(X, emb_E, emb_R):
    hs = X[:, 0].reshape(B // 128, 128)
    ls = X[:, 1].reshape(B // 128, 128)
    ts = X[:, 2].reshape(B // 128, 128)
    return _transe(hs, ls, ts, emb_E, emb_R).reshape(-1, 1)


# speedup vs baseline: 1.0090x; 1.0090x over previous
import functools

import jax
import jax.numpy as jnp
from jax import lax
from jax.experimental import pallas as pl
from jax.experimental.pallas import tpu as pltpu
from jax.experimental.pallas import tpu_sc as plsc

B = 16384
K = 64
NUM_WORKERS = 32
TRIPLES_PER_WORKER = B // NUM_WORKERS
CHUNKS = TRIPLES_PER_WORKER // 128
GROUPS = TRIPLES_PER_WORKER // 16


def _sqrt16(x):
    i = plsc.bitcast(x, jnp.int32)
    magic = jnp.full((16,), 0x5F3759DF, dtype=jnp.int32)
    y = plsc.bitcast(magic - (i >> 1), jnp.float32)
    half = jnp.full((16,), 0.5, dtype=jnp.float32)
    threehalf = jnp.full((16,), 1.5, dtype=jnp.float32)
    hx = half * x
    for _ in range(3):
        y = y * (threehalf - hx * y * y)
    return x * y


def _body(hs, ls, ts, emb_E, emb_R, out,
          idx_h, idx_l, idx_t, h_rows, l_rows, t_rows, out_v, sem):
    wid = lax.axis_index("s") * 2 + lax.axis_index("c")
    base_chunk = wid * CHUNKS

    pltpu.sync_copy(hs.at[pl.ds(base_chunk, CHUNKS)], idx_h)
    pltpu.sync_copy(ls.at[pl.ds(base_chunk, CHUNKS)], idx_l)
    pltpu.sync_copy(ts.at[pl.ds(base_chunk, CHUNKS)], idx_t)

    copies = []
    for c in range(CHUNKS):
        dst = pl.ds(c * 128, 128)
        copies.append(pltpu.async_copy(emb_E.at[idx_h.at[c]], h_rows.at[dst], sem))
        copies.append(pltpu.async_copy(emb_R.at[idx_l.at[c]], l_rows.at[dst], sem))
        copies.append(pltpu.async_copy(emb_E.at[idx_t.at[c]], t_rows.at[dst], sem))
    for cp in copies:
        cp.wait()

    lane = lax.iota(jnp.int32, 16)

    UNROLL = 8

    def group_body(g, carry):
        row = g * 16 + lane

        def j_body(jc, accs):
            accs = list(accs)
            jbase = jc * UNROLL
            for u in range(UNROLL):
                col = jnp.full((16,), jbase + u, dtype=jnp.int32)
                hv = plsc.load_gather(h_rows, [row, col])
                lv = plsc.load_gather(l_rows, [row, col])
                tv = plsc.load_gather(t_rows, [row, col])
                d = hv + lv - tv
                accs[u % 4] = accs[u % 4] + d * d
            return tuple(accs)

        zero = jnp.zeros((16,), jnp.float32)
        a0, a1, a2, a3 = lax.fori_loop(
            0, K // UNROLL, j_body, (zero, zero, zero, zero))
        acc = (a0 + a1) + (a2 + a3)
        plsc.store_scatter(out_v, [row], _sqrt16(acc))
        return carry

    lax.fori_loop(0, GROUPS, group_body, 0)

    pltpu.sync_copy(out_v, out.at[pl.ds(wid * TRIPLES_PER_WORKER,
                                        TRIPLES_PER_WORKER)])


@functools.partial(jax.jit, donate_argnums=())
def _transe(hs, ls, ts, emb_E, emb_R):
    mesh = plsc.VectorSubcoreMesh(core_axis_name="c", subcore_axis_name="s")
    f = functools.partial(
        pl.kernel,
        out_type=jax.ShapeDtypeStruct((B,), jnp.float32),
        mesh=mesh,
        compiler_params=pltpu.CompilerParams(
            needs_layout_passes=False, use_tc_tiling_on_sc=False),
        scratch_types=[
            pltpu.VMEM((CHUNKS, 128), jnp.int32),
            pltpu.VMEM((CHUNKS, 128), jnp.int32),
            pltpu.VMEM((CHUNKS, 128), jnp.int32),
            pltpu.VMEM((TRIPLES_PER_WORKER, K), jnp.float32),
            pltpu.VMEM((TRIPLES_PER_WORKER, K), jnp.float32),
            pltpu.VMEM((TRIPLES_PER_WORKER, K), jnp.float32),
            pltpu.VMEM((TRIPLES_PER_WORKER,), jnp.float32),
            pltpu.SemaphoreType.DMA,
        ],
    )(_body)
    return f(hs, ls, ts, emb_E, emb_R)


def kernel(X, emb_E, emb_R):
    hs = X[:, 0].reshape(B // 128, 128)
    ls = X[:, 1].reshape(B // 128, 128)
    ts = X[:, 2].reshape(B // 128, 128)
    return _transe(hs, ls, ts, emb_E, emb_R).reshape(-1, 1)
